# baseline (speedup 1.0000x reference)
import functools

import jax
import jax.numpy as jnp
from jax import lax
from jax.experimental import pallas as pl
from jax.experimental.pallas import tpu as pltpu
from jax.experimental.pallas import tpu_sc as plsc

NUM_FIXED = 900000
NUM_TRAIN = 100000
EMBED_DIM = 32
NC = 2
NS = 16
L = 16
NW = NC * NS

CHUNK = 800
GB = 80
GBLK = CHUNK // GB
BLK = 128
NBLK = -(-CHUNK // BLK)
NSLOT = 2
STAGE = NBLK * BLK + L


@jax.jit
def _lookup(inp, fixed_weights, trainable_weight):
    nb, nh = inp.shape
    idx = inp.reshape(-1)
    b_total = nb * nh
    assert b_total % NW == 0
    bpw = b_total // NW
    assert bpw % (CHUNK * NSLOT) == 0 and CHUNK % nh == 0
    nchunk = bpw // CHUNK
    rows_per_chunk = CHUNK // nh
    trash_slot = STAGE - L

    mesh = plsc.VectorSubcoreMesh(
        core_axis_name="c", subcore_axis_name="s", num_cores=NC, num_subcores=NS
    )

    slot_types = [
        pltpu.VMEM((CHUNK,), jnp.int32),
        pltpu.VMEM((GBLK, GB), jnp.int32),
        pltpu.VMEM((CHUNK + 8, EMBED_DIM), jnp.float32),
        pltpu.VMEM((STAGE,), jnp.int32),
        pltpu.VMEM((STAGE,), jnp.int32),
        pltpu.VMEM((BLK,), jnp.int32),
        pltpu.VMEM((BLK, EMBED_DIM), jnp.float32),
        pltpu.SemaphoreType.DMA,
        pltpu.SemaphoreType.DMA,
        pltpu.SemaphoreType.DMA,
    ]

    @functools.partial(
        pl.kernel,
        out_type=jax.ShapeDtypeStruct((nb, nh, EMBED_DIM), jnp.float32),
        mesh=mesh,
        scratch_types=slot_types * NSLOT,
        compiler_params=pltpu.CompilerParams(
            needs_layout_passes=False, use_tc_tiling_on_sc=False),
    )
    def k(idx_hbm, fixed_hbm, train_hbm, out_hbm, *scratch):
        ns = len(slot_types)
        slots = tuple(scratch[i * ns:(i + 1) * ns] for i in range(NSLOT))
        wid = lax.axis_index("s") * NC + lax.axis_index("c")
        base = wid * bpw
        iota = lax.iota(jnp.int32, L)

        def build_tblk(tst, tblk, b2):
            for g in range(BLK // L):
                v = tst[pl.ds(b2 * BLK + g * L, L)]
                tblk[pl.ds(g * L, L)] = jnp.clip(v, 0, NUM_TRAIN - 1)

        def start_chunk(c, b):
            idx_v, clamp_v, rows_v, tst, pst, tblk, tline, gsem, tsem, _ = \
                slots[b]
            cbase = base + c * CHUNK
            pltpu.sync_copy(idx_hbm.at[pl.ds(cbase, CHUNK)], idx_v)
            m_c = jnp.int32(0)
            for j in range(GBLK):
                def grp(g, m_c):
                    off = j * GB + g * L
                    v = idx_v[pl.ds(off, L)]
                    m = v >= NUM_FIXED
                    clamp_v[j, pl.ds(g * L, L)] = jnp.where(m, 0, v)
                    mi = m.astype(jnp.int32)
                    dest = jnp.where(
                        m, m_c + plsc.cumsum(mi) - 1, trash_slot + iota)
                    plsc.store_scatter(tst, [dest], v - NUM_FIXED)
                    plsc.store_scatter(pst, [dest], off + iota)
                    return m_c + jnp.sum(mi)
                m_c = lax.fori_loop(0, GB // L, grp, m_c)
                pltpu.async_copy(
                    fixed_hbm.at[clamp_v.at[j]],
                    rows_v.at[pl.ds(j * GB, GB)],
                    gsem,
                )
            build_tblk(tst, tblk, 0)
            pltpu.async_copy(train_hbm.at[tblk], tline, tsem)
            return m_c

        def merge_block(b, b2, m_c):
            _, _, rows_v, _, pst, _, tline, _, _, _ = slots[b]
            for g in range(BLK // L):
                ent = b2 * BLK + g * L + iota
                valid = ent < m_c
                posv = pst[pl.ds(b2 * BLK + g * L, L)]
                dst_row = jnp.where(valid, posv, CHUNK)
                src_row = g * L + iota
                for e in range(EMBED_DIM):
                    col = jnp.zeros((L,), jnp.int32) + e
                    vals = plsc.load_gather(tline, [src_row, col])
                    plsc.store_scatter(rows_v, [dst_row, col], vals)

        def finish_chunk(c, b, m_c):
            idx_v, clamp_v, rows_v, tst, pst, tblk, tline, gsem, tsem, wsem = \
                slots[b]
            pltpu.make_async_copy(
                fixed_hbm.at[pl.ds(0, CHUNK)], rows_v.at[pl.ds(0, CHUNK)],
                gsem).wait()
            pltpu.make_async_copy(train_hbm.at[tblk], tline, tsem).wait()
            merge_block(b, 0, m_c)

            def extra(b2, _):
                build_tblk(tst, tblk, b2)
                pltpu.async_copy(train_hbm.at[tblk], tline, tsem).wait()
                merge_block(b, b2, m_c)
                return 0
            nblk_c = (m_c + (BLK - 1)) // BLK
            lax.fori_loop(1, nblk_c, extra, 0)

            crow = (base + c * CHUNK) // nh
            for r in range(rows_per_chunk):
                pltpu.async_copy(
                    rows_v.at[pl.ds(r * nh, nh)],
                    out_hbm.at[crow + r],
                    wsem,
                )

        def drain_writeback(b):
            rows_v, wsem = slots[b][2], slots[b][9]
            for r in range(rows_per_chunk):
                pltpu.make_async_copy(
                    rows_v.at[pl.ds(0, nh)], out_hbm.at[0], wsem).wait()

        m0 = start_chunk(jnp.int32(0), 0)
        m1 = start_chunk(jnp.int32(1), 1)

        def g_body(g, carry):
            ms = list(carry)
            for b in range(NSLOT):
                c = NSLOT * g + b
                finish_chunk(c, b, ms[b])

                def prep(_):
                    drain_writeback(b)
                    return start_chunk(c + NSLOT, b)
                ms[b] = lax.cond(
                    c + NSLOT < nchunk, prep, lambda _: jnp.int32(0), 0)
            return tuple(ms)

        lax.fori_loop(0, nchunk // NSLOT, g_body, (m0, m1))
        for b in range(NSLOT):
            drain_writeback(b)

    return k(idx, fixed_weights, trainable_weight)


def kernel(inp, fixed_weights, trainable_weight):
    return _lookup(inp.astype(jnp.int32), fixed_weights, trainable_weight)
